# baseline (speedup 1.0000x reference)
import functools
import math

import jax
import jax.numpy as jnp
import numpy as np
from jax import lax
from jax.experimental import pallas as pl
from jax.experimental.pallas import tpu as pltpu
from jax.sharding import Mesh, PartitionSpec as P

_LOG2E = 1.4426950408889634


def _prep_kernel(zi_ref, zj_ref, reps_ref, pos_ref, sd_ref, *, scale2, inv_t):
    zi = zi_ref[...]
    zj = zj_ref[...]
    zi_n = zi * lax.rsqrt(jnp.maximum(jnp.sum(zi * zi, axis=-1, keepdims=True),
                                      1e-24))
    zj_n = zj * lax.rsqrt(jnp.maximum(jnp.sum(zj * zj, axis=-1, keepdims=True),
                                      1e-24))
    pos_ref[...] = jnp.float32(inv_t) * jnp.sum(zi_n * zj_n, axis=-1,
                                                keepdims=True)
    a = (zi_n * jnp.float32(scale2)).astype(jnp.bfloat16)
    b = (zj_n * jnp.float32(scale2)).astype(jnp.bfloat16)
    reps_ref[0] = a
    reps_ref[1] = b
    af = a.astype(jnp.float32)
    bf = b.astype(jnp.float32)
    sd_ref[0] = jnp.sum(af * af, axis=-1, keepdims=True)
    sd_ref[1] = jnp.sum(bf * bf, axis=-1, keepdims=True)


def _sumexp_kernel(q_ref, kt_ref, out_ref, acc_ref, *, bk, acc_w, sub):
    kc = pl.program_id(1)

    @pl.when(kc == 0)
    def _():
        acc_ref[...] = jnp.zeros_like(acc_ref)

    q = q_ref[...]
    part0 = None
    part1 = None
    for c in range(bk // sub):
        start = pl.multiple_of(kc * bk + c * sub, sub)
        s = jnp.dot(q, kt_ref[:, pl.ds(start, sub)],
                    preferred_element_type=jnp.float32)
        p = jnp.exp2(s)
        for j in range(sub // acc_w):
            chunk = p[:, j * acc_w:(j + 1) * acc_w]
            if j % 2 == 0:
                part0 = chunk if part0 is None else part0 + chunk
            else:
                part1 = chunk if part1 is None else part1 + chunk
    part = part0 if part1 is None else part0 + part1
    acc_ref[...] += part

    @pl.when(kc == pl.num_programs(1) - 1)
    def _():
        out_ref[...] = acc_ref[...]


def _round_up(x, mult):
    return (x + mult - 1) // mult * mult


def _pick_block(total, candidates):
    for c in candidates:
        if c <= total and total % c == 0:
            return c
    return total


def _ntxent_local(z_i, z_j, *, inv_t, scale2, ndev):
    n_loc, d = z_i.shape
    m_loc = 2 * n_loc
    m = ndev * m_loc

    d_pad = max(128, _round_up(d, 128))
    if d_pad != d:
        z_i = jnp.pad(z_i, ((0, 0), (0, d_pad - d)))
        z_j = jnp.pad(z_j, ((0, 0), (0, d_pad - d)))

    bn = _pick_block(n_loc, (256, 128, 64, 32, 16, 8))

    reps, pos, sd = pl.pallas_call(
        functools.partial(_prep_kernel, scale2=scale2, inv_t=inv_t),
        grid=(n_loc // bn,),
        in_specs=[pl.BlockSpec((bn, d_pad), lambda i: (i, 0)),
                  pl.BlockSpec((bn, d_pad), lambda i: (i, 0))],
        out_specs=(pl.BlockSpec((2, bn, d_pad), lambda i: (0, i, 0)),
                   pl.BlockSpec((bn, 1), lambda i: (i, 0)),
                   pl.BlockSpec((2, bn, 1), lambda i: (0, i, 0))),
        out_shape=(jax.ShapeDtypeStruct((2, n_loc, d_pad), jnp.bfloat16),
                   jax.ShapeDtypeStruct((n_loc, 1), jnp.float32),
                   jax.ShapeDtypeStruct((2, n_loc, 1), jnp.float32)),
        compiler_params=pltpu.CompilerParams(
            dimension_semantics=("parallel",),
            vmem_limit_bytes=48 * 1024 * 1024),
    )(z_i, z_j)

    q = reps.reshape(m_loc, d_pad)
    ktl = q.T
    if ndev > 1:
        kt = lax.all_gather(ktl, "x", axis=1, tiled=True)
    else:
        kt = ktl

    bq = _pick_block(m_loc, (512, 256, 128, 64, 32, 16, 8))
    bk = _pick_block(m, (2048, 1024, 512, 256, 128))
    sub = min(bk, 512)
    acc_w = 128 if bk % 128 == 0 else bk

    est2 = (2 * m * d_pad * 2
            + 2 * bq * d_pad * 2
            + 3 * bq * acc_w * 4
            + 4 * bq * sub * 4)
    cost = pl.CostEstimate(flops=2 * m_loc * m * d_pad,
                           transcendentals=m_loc * m,
                           bytes_accessed=(m_loc + m) * d_pad * 2
                           + m_loc * acc_w * 4)

    part = pl.pallas_call(
        functools.partial(_sumexp_kernel, bk=bk, acc_w=acc_w, sub=sub),
        grid=(m_loc // bq, m // bk),
        in_specs=[pl.BlockSpec((bq, d_pad), lambda qr, kc: (qr, 0)),
                  pl.BlockSpec((d_pad, m), lambda qr, kc: (0, 0))],
        out_specs=pl.BlockSpec((bq, acc_w), lambda qr, kc: (qr, 0)),
        out_shape=jax.ShapeDtypeStruct((m_loc, acc_w), jnp.float32),
        scratch_shapes=[pltpu.VMEM((bq, acc_w), jnp.float32)],
        compiler_params=pltpu.CompilerParams(
            dimension_semantics=("parallel", "arbitrary"),
            vmem_limit_bytes=min(64 * 1024 * 1024,
                                 max(32 * 1024 * 1024, 2 * est2))),
        cost_estimate=cost,
    )(q, kt)

    s_row = jnp.sum(part, axis=-1)
    denom = s_row - jnp.exp2(sd.reshape(m_loc))
    lse = jnp.log(denom)
    partial = jnp.sum(lse) - 2.0 * jnp.sum(pos)
    if ndev > 1:
        partial = lax.psum(partial, "x")
    return partial / jnp.float32(m)


def kernel(z_i, z_j, temperature=0.5):
    assert z_i.shape == z_j.shape and z_i.ndim == 2
    n, d = z_i.shape
    inv_t = 1.0 / float(temperature)
    scale2 = math.sqrt(inv_t * _LOG2E)

    devs = jax.devices()
    ndev = 2 if (len(devs) >= 2 and n % 16 == 0) else 1
    body = functools.partial(_ntxent_local, inv_t=inv_t, scale2=scale2,
                             ndev=ndev)
    if ndev == 1:
        return body(z_i, z_j)
    mesh = Mesh(np.array(devs[:ndev]), ("x",))
    f = jax.shard_map(body, mesh=mesh,
                      in_specs=(P("x", None), P("x", None)),
                      out_specs=P(), check_vma=False)
    return f(z_i, z_j)


# speedup vs baseline: 2.2967x; 2.2967x over previous
import functools
import math

import jax
import jax.numpy as jnp
from jax import lax
from jax.experimental import pallas as pl
from jax.experimental.pallas import tpu as pltpu

_LOG2E = 1.4426950408889634


def _prep_kernel(zi_ref, zj_ref, reps_ref, pos_ref, sd_ref, *, scale2, inv_t):
    zi = zi_ref[...]
    zj = zj_ref[...]
    zi_n = zi * lax.rsqrt(jnp.maximum(jnp.sum(zi * zi, axis=-1, keepdims=True),
                                      1e-24))
    zj_n = zj * lax.rsqrt(jnp.maximum(jnp.sum(zj * zj, axis=-1, keepdims=True),
                                      1e-24))
    pos_ref[...] = jnp.float32(inv_t) * jnp.sum(zi_n * zj_n, axis=-1,
                                                keepdims=True)
    a = (zi_n * jnp.float32(scale2)).astype(jnp.bfloat16)
    b = (zj_n * jnp.float32(scale2)).astype(jnp.bfloat16)
    reps_ref[0] = a
    reps_ref[1] = b
    af = a.astype(jnp.float32)
    bf = b.astype(jnp.float32)
    sd_ref[0] = jnp.sum(af * af, axis=-1, keepdims=True)
    sd_ref[1] = jnp.sum(bf * bf, axis=-1, keepdims=True)


def _sumexp_kernel(q_ref, kt_ref, out_ref, acc_ref, *, bk, acc_w, sub):
    kc = pl.program_id(1)

    @pl.when(kc == 0)
    def _():
        acc_ref[...] = jnp.zeros_like(acc_ref)

    q = q_ref[...]
    part0 = None
    part1 = None
    for c in range(bk // sub):
        start = pl.multiple_of(kc * bk + c * sub, sub)
        s = jnp.dot(q, kt_ref[:, pl.ds(start, sub)],
                    preferred_element_type=jnp.float32)
        p = jnp.exp2(s)
        for j in range(sub // acc_w):
            chunk = p[:, j * acc_w:(j + 1) * acc_w]
            if j % 2 == 0:
                part0 = chunk if part0 is None else part0 + chunk
            else:
                part1 = chunk if part1 is None else part1 + chunk
    part = part0 if part1 is None else part0 + part1
    acc_ref[...] += part

    @pl.when(kc == pl.num_programs(1) - 1)
    def _():
        out_ref[...] = acc_ref[...]


def _round_up(x, mult):
    return (x + mult - 1) // mult * mult


def _pick_block(total, candidates):
    for c in candidates:
        if c <= total and total % c == 0:
            return c
    return total


def kernel(z_i, z_j, temperature=0.5):
    assert z_i.shape == z_j.shape and z_i.ndim == 2
    n, d = z_i.shape
    m = 2 * n
    inv_t = 1.0 / float(temperature)
    scale2 = math.sqrt(inv_t * _LOG2E)

    d_pad = max(128, _round_up(d, 128))
    if d_pad != d:
        z_i = jnp.pad(z_i, ((0, 0), (0, d_pad - d)))
        z_j = jnp.pad(z_j, ((0, 0), (0, d_pad - d)))

    bn = _pick_block(n, (256, 128, 64, 32, 16, 8))

    reps, pos, sd = pl.pallas_call(
        functools.partial(_prep_kernel, scale2=scale2, inv_t=inv_t),
        grid=(n // bn,),
        in_specs=[pl.BlockSpec((bn, d_pad), lambda i: (i, 0)),
                  pl.BlockSpec((bn, d_pad), lambda i: (i, 0))],
        out_specs=(pl.BlockSpec((2, bn, d_pad), lambda i: (0, i, 0)),
                   pl.BlockSpec((bn, 1), lambda i: (i, 0)),
                   pl.BlockSpec((2, bn, 1), lambda i: (0, i, 0))),
        out_shape=(jax.ShapeDtypeStruct((2, n, d_pad), jnp.bfloat16),
                   jax.ShapeDtypeStruct((n, 1), jnp.float32),
                   jax.ShapeDtypeStruct((2, n, 1), jnp.float32)),
        compiler_params=pltpu.CompilerParams(
            dimension_semantics=("parallel",),
            vmem_limit_bytes=48 * 1024 * 1024),
    )(z_i, z_j)

    q = reps.reshape(m, d_pad)
    kt = q.T

    bq = _pick_block(m, (512, 256, 128, 64, 32, 16, 8))
    bk = _pick_block(m, (4096, 2048, 1024, 512, 256, 128))
    sub = min(bk, 512)
    acc_w = 128 if bk % 128 == 0 else bk

    est2 = (2 * m * d_pad * 2
            + 2 * bq * d_pad * 2
            + 3 * bq * acc_w * 4
            + 4 * bq * sub * 4)
    cost = pl.CostEstimate(flops=2 * m * m * d_pad,
                           transcendentals=m * m,
                           bytes_accessed=2 * m * d_pad * 2 + m * acc_w * 4)

    part = pl.pallas_call(
        functools.partial(_sumexp_kernel, bk=bk, acc_w=acc_w, sub=sub),
        grid=(m // bq, m // bk),
        in_specs=[pl.BlockSpec((bq, d_pad), lambda qr, kc: (qr, 0)),
                  pl.BlockSpec((d_pad, m), lambda qr, kc: (0, 0))],
        out_specs=pl.BlockSpec((bq, acc_w), lambda qr, kc: (qr, 0)),
        out_shape=jax.ShapeDtypeStruct((m, acc_w), jnp.float32),
        scratch_shapes=[pltpu.VMEM((bq, acc_w), jnp.float32)],
        compiler_params=pltpu.CompilerParams(
            dimension_semantics=("parallel", "arbitrary"),
            vmem_limit_bytes=min(64 * 1024 * 1024,
                                 max(32 * 1024 * 1024, 2 * est2))),
        cost_estimate=cost,
    )(q, kt)

    s_row = jnp.sum(part, axis=-1)
    denom = s_row - jnp.exp2(sd.reshape(m))
    lse = jnp.log(denom)
    return (jnp.sum(lse) - 2.0 * jnp.sum(pos)) / jnp.float32(m)


# speedup vs baseline: 2.4508x; 1.0671x over previous
import functools
import math

import jax
import jax.numpy as jnp
from jax import lax
from jax.experimental import pallas as pl
from jax.experimental.pallas import tpu as pltpu

_LOG2E = 1.4426950408889634


def _prep_kernel(zi_ref, zj_ref, reps_ref, pos_ref, sd_ref, *, scale2, inv_t):
    zi = zi_ref[...]
    zj = zj_ref[...]
    zi_n = zi * lax.rsqrt(jnp.maximum(jnp.sum(zi * zi, axis=-1, keepdims=True),
                                      1e-24))
    zj_n = zj * lax.rsqrt(jnp.maximum(jnp.sum(zj * zj, axis=-1, keepdims=True),
                                      1e-24))
    pos_ref[...] = jnp.float32(inv_t) * jnp.sum(zi_n * zj_n, axis=-1,
                                                keepdims=True)
    a = (zi_n * jnp.float32(scale2)).astype(jnp.bfloat16)
    b = (zj_n * jnp.float32(scale2)).astype(jnp.bfloat16)
    reps_ref[0] = a
    reps_ref[1] = b
    af = a.astype(jnp.float32)
    bf = b.astype(jnp.float32)
    sd_ref[0] = jnp.sum(af * af, axis=-1, keepdims=True)
    sd_ref[1] = jnp.sum(bf * bf, axis=-1, keepdims=True)


def _sumexp_kernel(q_ref, kt_ref, out_ref, acc_ref, *, bk, acc_w, sub):
    kc = pl.program_id(1)

    @pl.when(kc == 0)
    def _():
        acc_ref[...] = jnp.zeros_like(acc_ref)

    q = q_ref[...]
    part0 = None
    part1 = None
    for c in range(bk // sub):
        start = pl.multiple_of(kc * bk + c * sub, sub)
        s = jnp.dot(q, kt_ref[:, pl.ds(start, sub)],
                    preferred_element_type=jnp.float32)
        p = jnp.exp2(s)
        for j in range(sub // acc_w):
            chunk = p[:, j * acc_w:(j + 1) * acc_w]
            if j % 2 == 0:
                part0 = chunk if part0 is None else part0 + chunk
            else:
                part1 = chunk if part1 is None else part1 + chunk
    part = part0 if part1 is None else part0 + part1
    acc_ref[...] += part

    @pl.when(kc == pl.num_programs(1) - 1)
    def _():
        out_ref[...] = acc_ref[...]


def _round_up(x, mult):
    return (x + mult - 1) // mult * mult


def _pick_block(total, candidates):
    for c in candidates:
        if c <= total and total % c == 0:
            return c
    return total


def kernel(z_i, z_j, temperature=0.5):
    assert z_i.shape == z_j.shape and z_i.ndim == 2
    n, d = z_i.shape
    m = 2 * n
    inv_t = 1.0 / float(temperature)
    scale2 = math.sqrt(inv_t * _LOG2E)

    d_pad = max(128, _round_up(d, 128))
    if d_pad != d:
        z_i = jnp.pad(z_i, ((0, 0), (0, d_pad - d)))
        z_j = jnp.pad(z_j, ((0, 0), (0, d_pad - d)))

    bn = _pick_block(n, (256, 128, 64, 32, 16, 8))

    reps, pos, sd = pl.pallas_call(
        functools.partial(_prep_kernel, scale2=scale2, inv_t=inv_t),
        grid=(n // bn,),
        in_specs=[pl.BlockSpec((bn, d_pad), lambda i: (i, 0)),
                  pl.BlockSpec((bn, d_pad), lambda i: (i, 0))],
        out_specs=(pl.BlockSpec((2, bn, d_pad), lambda i: (0, i, 0)),
                   pl.BlockSpec((bn, 1), lambda i: (i, 0)),
                   pl.BlockSpec((2, bn, 1), lambda i: (0, i, 0))),
        out_shape=(jax.ShapeDtypeStruct((2, n, d_pad), jnp.bfloat16),
                   jax.ShapeDtypeStruct((n, 1), jnp.float32),
                   jax.ShapeDtypeStruct((2, n, 1), jnp.float32)),
        compiler_params=pltpu.CompilerParams(
            dimension_semantics=("parallel",),
            vmem_limit_bytes=48 * 1024 * 1024),
    )(z_i, z_j)

    q = reps.reshape(m, d_pad)
    kt = q.T

    bq = _pick_block(m, (512, 256, 128, 64, 32, 16, 8))
    bk = _pick_block(m, (8192, 4096, 2048, 1024, 512, 256, 128))
    sub = min(bk, 512)
    acc_w = 128 if bk % 128 == 0 else bk

    est2 = (2 * m * d_pad * 2
            + 2 * bq * d_pad * 2
            + 3 * bq * acc_w * 4
            + 4 * bq * sub * 4)
    cost = pl.CostEstimate(flops=2 * m * m * d_pad,
                           transcendentals=m * m,
                           bytes_accessed=2 * m * d_pad * 2 + m * acc_w * 4)

    part = pl.pallas_call(
        functools.partial(_sumexp_kernel, bk=bk, acc_w=acc_w, sub=sub),
        grid=(m // bq, m // bk),
        in_specs=[pl.BlockSpec((bq, d_pad), lambda qr, kc: (qr, 0)),
                  pl.BlockSpec((d_pad, m), lambda qr, kc: (0, 0))],
        out_specs=pl.BlockSpec((bq, acc_w), lambda qr, kc: (qr, 0)),
        out_shape=jax.ShapeDtypeStruct((m, acc_w), jnp.float32),
        scratch_shapes=[pltpu.VMEM((bq, acc_w), jnp.float32)],
        compiler_params=pltpu.CompilerParams(
            dimension_semantics=("parallel", "arbitrary"),
            vmem_limit_bytes=min(64 * 1024 * 1024,
                                 max(32 * 1024 * 1024, 2 * est2))),
        cost_estimate=cost,
    )(q, kt)

    s_row = jnp.sum(part, axis=-1)
    denom = s_row - jnp.exp2(sd.reshape(m))
    lse = jnp.log(denom)
    return (jnp.sum(lse) - 2.0 * jnp.sum(pos)) / jnp.float32(m)


# speedup vs baseline: 2.5562x; 1.0430x over previous
import functools
import math

import jax
import jax.numpy as jnp
from jax import lax
from jax.experimental import pallas as pl
from jax.experimental.pallas import tpu as pltpu

_LOG2E = 1.4426950408889634


def _prep_kernel(zi_ref, zj_ref, reps_ref, pos_ref, sd_ref, *, scale2, inv_t):
    zi = zi_ref[...]
    zj = zj_ref[...]
    zi_n = zi * lax.rsqrt(jnp.maximum(jnp.sum(zi * zi, axis=-1, keepdims=True),
                                      1e-24))
    zj_n = zj * lax.rsqrt(jnp.maximum(jnp.sum(zj * zj, axis=-1, keepdims=True),
                                      1e-24))
    pos_ref[...] = jnp.float32(inv_t) * jnp.sum(zi_n * zj_n, axis=-1,
                                                keepdims=True)
    a = (zi_n * jnp.float32(scale2)).astype(jnp.bfloat16)
    b = (zj_n * jnp.float32(scale2)).astype(jnp.bfloat16)
    reps_ref[0] = a
    reps_ref[1] = b
    af = a.astype(jnp.float32)
    bf = b.astype(jnp.float32)
    sd_ref[0] = jnp.sum(af * af, axis=-1, keepdims=True)
    sd_ref[1] = jnp.sum(bf * bf, axis=-1, keepdims=True)


def _sumexp_kernel(q_ref, kt_ref, out_ref, acc_ref, *, bk, acc_w, sub):
    kc = pl.program_id(1)

    @pl.when(kc == 0)
    def _():
        acc_ref[...] = jnp.zeros_like(acc_ref)

    q = q_ref[...]
    part0 = None
    part1 = None
    for c in range(bk // sub):
        start = pl.multiple_of(kc * bk + c * sub, sub)
        s = jnp.dot(q, kt_ref[:, pl.ds(start, sub)],
                    preferred_element_type=jnp.float32)
        p = jnp.exp2(s)
        for j in range(sub // acc_w):
            chunk = p[:, j * acc_w:(j + 1) * acc_w]
            if j % 2 == 0:
                part0 = chunk if part0 is None else part0 + chunk
            else:
                part1 = chunk if part1 is None else part1 + chunk
    part = part0 if part1 is None else part0 + part1
    acc_ref[...] += part

    @pl.when(kc == pl.num_programs(1) - 1)
    def _():
        out_ref[...] = acc_ref[...]


def _round_up(x, mult):
    return (x + mult - 1) // mult * mult


def _pick_block(total, candidates):
    for c in candidates:
        if c <= total and total % c == 0:
            return c
    return total


def kernel(z_i, z_j, temperature=0.5):
    assert z_i.shape == z_j.shape and z_i.ndim == 2
    n, d = z_i.shape
    m = 2 * n
    inv_t = 1.0 / float(temperature)
    scale2 = math.sqrt(inv_t * _LOG2E)

    d_pad = max(128, _round_up(d, 128))
    if d_pad != d:
        z_i = jnp.pad(z_i, ((0, 0), (0, d_pad - d)))
        z_j = jnp.pad(z_j, ((0, 0), (0, d_pad - d)))

    bn = _pick_block(n, (256, 128, 64, 32, 16, 8))

    reps, pos, sd = pl.pallas_call(
        functools.partial(_prep_kernel, scale2=scale2, inv_t=inv_t),
        grid=(n // bn,),
        in_specs=[pl.BlockSpec((bn, d_pad), lambda i: (i, 0)),
                  pl.BlockSpec((bn, d_pad), lambda i: (i, 0))],
        out_specs=(pl.BlockSpec((2, bn, d_pad), lambda i: (0, i, 0)),
                   pl.BlockSpec((bn, 1), lambda i: (i, 0)),
                   pl.BlockSpec((2, bn, 1), lambda i: (0, i, 0))),
        out_shape=(jax.ShapeDtypeStruct((2, n, d_pad), jnp.bfloat16),
                   jax.ShapeDtypeStruct((n, 1), jnp.float32),
                   jax.ShapeDtypeStruct((2, n, 1), jnp.float32)),
        compiler_params=pltpu.CompilerParams(
            dimension_semantics=("parallel",),
            vmem_limit_bytes=48 * 1024 * 1024),
    )(z_i, z_j)

    q = reps.reshape(m, d_pad)
    kt = q.T

    bq = _pick_block(m, (512, 256, 128, 64, 32, 16, 8))
    bk = _pick_block(m, (16384, 8192, 4096, 2048, 1024, 512, 256, 128))
    sub = min(bk, 512)
    acc_w = 128 if bk % 128 == 0 else bk

    est2 = (2 * m * d_pad * 2
            + 2 * bq * d_pad * 2
            + 3 * bq * acc_w * 4
            + 4 * bq * sub * 4)
    cost = pl.CostEstimate(flops=2 * m * m * d_pad,
                           transcendentals=m * m,
                           bytes_accessed=2 * m * d_pad * 2 + m * acc_w * 4)

    part = pl.pallas_call(
        functools.partial(_sumexp_kernel, bk=bk, acc_w=acc_w, sub=sub),
        grid=(m // bq, m // bk),
        in_specs=[pl.BlockSpec((bq, d_pad), lambda qr, kc: (qr, 0)),
                  pl.BlockSpec((d_pad, m), lambda qr, kc: (0, 0))],
        out_specs=pl.BlockSpec((bq, acc_w), lambda qr, kc: (qr, 0)),
        out_shape=jax.ShapeDtypeStruct((m, acc_w), jnp.float32),
        scratch_shapes=[pltpu.VMEM((bq, acc_w), jnp.float32)],
        compiler_params=pltpu.CompilerParams(
            dimension_semantics=("parallel", "arbitrary"),
            vmem_limit_bytes=min(64 * 1024 * 1024,
                                 max(32 * 1024 * 1024, 2 * est2))),
        cost_estimate=cost,
    )(q, kt)

    s_row = jnp.sum(part, axis=-1)
    denom = s_row - jnp.exp2(sd.reshape(m))
    lse = jnp.log(denom)
    return (jnp.sum(lse) - 2.0 * jnp.sum(pos)) / jnp.float32(m)


# speedup vs baseline: 2.9783x; 1.1651x over previous
import functools
import math

import jax
import jax.numpy as jnp
from jax import lax
from jax.experimental import pallas as pl
from jax.experimental.pallas import tpu as pltpu

_LOG2E = 1.4426950408889634


def _prep_kernel(zi_ref, zj_ref, reps_ref, kti_ref, ktj_ref, pos_ref, sd_ref,
                 *, scale2, inv_t):
    zi = zi_ref[...]
    zj = zj_ref[...]
    zi_n = zi * lax.rsqrt(jnp.maximum(jnp.sum(zi * zi, axis=-1, keepdims=True),
                                      1e-24))
    zj_n = zj * lax.rsqrt(jnp.maximum(jnp.sum(zj * zj, axis=-1, keepdims=True),
                                      1e-24))
    pos_ref[...] = jnp.float32(inv_t) * jnp.sum(zi_n * zj_n, axis=-1,
                                                keepdims=True)
    a = (zi_n * jnp.float32(scale2)).astype(jnp.bfloat16)
    b = (zj_n * jnp.float32(scale2)).astype(jnp.bfloat16)
    reps_ref[0] = a
    reps_ref[1] = b
    kti_ref[...] = a.T
    ktj_ref[...] = b.T
    af = a.astype(jnp.float32)
    bf = b.astype(jnp.float32)
    sd_ref[0] = jnp.sum(af * af, axis=-1, keepdims=True)
    sd_ref[1] = jnp.sum(bf * bf, axis=-1, keepdims=True)


def _lse_kernel(q_ref, kti_ref, ktj_ref, sd_ref, lse_ref, *, sub, acc_w):
    q = q_ref[...]
    part0 = None
    part1 = None
    for kt_ref in (kti_ref, ktj_ref):
        width = kt_ref.shape[-1]
        for c in range(width // sub):
            s = jnp.dot(q, kt_ref[:, c * sub:(c + 1) * sub],
                        preferred_element_type=jnp.float32)
            p = jnp.exp2(s)
            for j in range(sub // acc_w):
                chunk = p[:, j * acc_w:(j + 1) * acc_w]
                if j % 2 == 0:
                    part0 = chunk if part0 is None else part0 + chunk
                else:
                    part1 = chunk if part1 is None else part1 + chunk
    part = part0 if part1 is None else part0 + part1
    s_row = jnp.sum(part, axis=-1, keepdims=True)
    lse_ref[...] = jnp.log(s_row - jnp.exp2(sd_ref[...]))


def _round_up(x, mult):
    return (x + mult - 1) // mult * mult


def _pick_block(total, candidates):
    for c in candidates:
        if c <= total and total % c == 0:
            return c
    return total


def kernel(z_i, z_j, temperature=0.5):
    assert z_i.shape == z_j.shape and z_i.ndim == 2
    n, d = z_i.shape
    m = 2 * n
    inv_t = 1.0 / float(temperature)
    scale2 = math.sqrt(inv_t * _LOG2E)

    d_pad = max(128, _round_up(d, 128))
    if d_pad != d:
        z_i = jnp.pad(z_i, ((0, 0), (0, d_pad - d)))
        z_j = jnp.pad(z_j, ((0, 0), (0, d_pad - d)))

    bn = _pick_block(n, (256, 128, 64, 32, 16, 8))

    reps, kti, ktj, pos, sd = pl.pallas_call(
        functools.partial(_prep_kernel, scale2=scale2, inv_t=inv_t),
        grid=(n // bn,),
        in_specs=[pl.BlockSpec((bn, d_pad), lambda i: (i, 0)),
                  pl.BlockSpec((bn, d_pad), lambda i: (i, 0))],
        out_specs=(pl.BlockSpec((2, bn, d_pad), lambda i: (0, i, 0)),
                   pl.BlockSpec((d_pad, bn), lambda i: (0, i)),
                   pl.BlockSpec((d_pad, bn), lambda i: (0, i)),
                   pl.BlockSpec((bn, 1), lambda i: (i, 0)),
                   pl.BlockSpec((2, bn, 1), lambda i: (0, i, 0))),
        out_shape=(jax.ShapeDtypeStruct((2, n, d_pad), jnp.bfloat16),
                   jax.ShapeDtypeStruct((d_pad, n), jnp.bfloat16),
                   jax.ShapeDtypeStruct((d_pad, n), jnp.bfloat16),
                   jax.ShapeDtypeStruct((n, 1), jnp.float32),
                   jax.ShapeDtypeStruct((2, n, 1), jnp.float32)),
        compiler_params=pltpu.CompilerParams(
            dimension_semantics=("parallel",),
            vmem_limit_bytes=48 * 1024 * 1024),
    )(z_i, z_j)

    q = reps.reshape(m, d_pad)
    sd_m = sd.reshape(m, 1)

    bq = _pick_block(m, (512, 256, 128, 64, 32, 16, 8))
    sub = min(n, 512)
    acc_w = 128 if sub % 128 == 0 else sub

    est2 = (2 * m * d_pad * 2
            + 2 * bq * d_pad * 2
            + 8 * bq * sub * 4)
    cost = pl.CostEstimate(flops=2 * m * m * d_pad,
                           transcendentals=m * m,
                           bytes_accessed=2 * m * d_pad * 2 + m * 4)

    lse = pl.pallas_call(
        functools.partial(_lse_kernel, sub=sub, acc_w=acc_w),
        grid=(m // bq,),
        in_specs=[pl.BlockSpec((bq, d_pad), lambda i: (i, 0)),
                  pl.BlockSpec((d_pad, n), lambda i: (0, 0)),
                  pl.BlockSpec((d_pad, n), lambda i: (0, 0)),
                  pl.BlockSpec((bq, 1), lambda i: (i, 0))],
        out_specs=pl.BlockSpec((bq, 1), lambda i: (i, 0)),
        out_shape=jax.ShapeDtypeStruct((m, 1), jnp.float32),
        compiler_params=pltpu.CompilerParams(
            dimension_semantics=("arbitrary",),
            vmem_limit_bytes=min(64 * 1024 * 1024,
                                 max(32 * 1024 * 1024, 2 * est2))),
        cost_estimate=cost,
    )(q, kti, ktj, sd_m)

    return (jnp.sum(lse) - 2.0 * jnp.sum(pos)) / jnp.float32(m)
